# Initial kernel scaffold; baseline (speedup 1.0000x reference)
import jax
import jax.numpy as jnp
from jax.experimental import pallas as pl


def kernel(x, edge_index, W, b):
    raise NotImplementedError("write your pallas kernel here")


# speedup vs baseline: 19.8404x; 19.8404x over previous
import jax
import jax.numpy as jnp
from jax import lax
from jax.experimental import pallas as pl
from jax.experimental.pallas import tpu as pltpu
from jax.experimental.pallas import tpu_sc as plsc

N = 10000
D = 128
K = 5000
E2 = 640000
NW = 32
CHUNK = 160
ECHUNK = E2 // NW
JPAD = 10240

_IB = 1000
_JB = 1024


def _scores_body(x_ref, w_ref, b_ref, out_ref):
    out_ref[...] = (
        jnp.dot(x_ref[...], w_ref[...], preferred_element_type=jnp.float32)
        + b_ref[0, 0]
    )


def _rank_body(scol_ref, srow_ref, out_ref):
    i = pl.program_id(0)
    j = pl.program_id(1)
    si = scol_ref[...]
    sj = srow_ref[...]
    ii = i * _IB + lax.broadcasted_iota(jnp.int32, (_IB, 1), 0)
    jj = j * _JB + lax.broadcasted_iota(jnp.int32, (1, _JB), 1)
    before = (sj > si) | ((sj == si) & (jj < ii))
    part = jnp.sum(before.astype(jnp.float32), axis=1, keepdims=True)

    @pl.when(j == 0)
    def _init():
        out_ref[...] = part

    @pl.when(j > 0)
    def _acc():
        out_ref[...] += part


def _sc_body(rank_hbm, edges_hbm, x_hbm, perm_hbm, pooled_hbm, eout_hbm,
             rank_v, perm_v, rows_v, ebuf, sem):
    wid = lax.axis_index("s") * 2 + lax.axis_index("c")
    lo = wid * CHUNK

    pltpu.sync_copy(rank_hbm, rank_v)

    def body_a(t, carry):
        r = rank_v[pl.ds(t * 16, 16)]
        m = (r >= lo) & (r < lo + CHUNK)
        vals = t * 16 + lax.iota(jnp.int32, 16)
        plsc.store_scatter(perm_v, [r - lo], vals, mask=m)
        return carry

    lax.fori_loop(0, N // 16, body_a, 0)

    pltpu.async_copy(
        x_hbm.at[perm_v.at[pl.ds(0, 80)]], rows_v.at[pl.ds(0, 80)], sem
    ).wait()
    pltpu.async_copy(
        x_hbm.at[perm_v.at[pl.ds(80, 80)]], rows_v.at[pl.ds(80, 80)], sem
    ).wait()

    @pl.when(wid < NW - 1)
    def _full():
        pltpu.sync_copy(perm_v, perm_hbm.at[pl.ds(lo, CHUNK)])
        pltpu.sync_copy(rows_v, pooled_hbm.at[pl.ds(lo, CHUNK)])

    @pl.when(wid == NW - 1)
    def _tail():
        pltpu.sync_copy(perm_v.at[pl.ds(0, 40)], perm_hbm.at[pl.ds(K - 40, 40)])
        pltpu.sync_copy(rows_v.at[pl.ds(0, 40)], pooled_hbm.at[pl.ds(K - 40, 40)])

    ebase = wid * ECHUNK
    pltpu.sync_copy(edges_hbm.at[pl.ds(ebase, ECHUNK)], ebuf)

    def body_c(t, carry):
        e = ebuf[pl.ds(t * 16, 16)]
        r = plsc.load_gather(rank_v, [e])
        ebuf[pl.ds(t * 16, 16)] = jnp.where(r < K, r, 0)
        return carry

    lax.fori_loop(0, ECHUNK // 16, body_c, 0)
    pltpu.sync_copy(ebuf, eout_hbm.at[pl.ds(ebase, ECHUNK)])


_sc_call = pl.kernel(
    _sc_body,
    out_type=(
        jax.ShapeDtypeStruct((K,), jnp.int32),
        jax.ShapeDtypeStruct((K, D), jnp.float32),
        jax.ShapeDtypeStruct((E2,), jnp.int32),
    ),
    mesh=plsc.VectorSubcoreMesh(core_axis_name="c", subcore_axis_name="s"),
    compiler_params=pltpu.CompilerParams(needs_layout_passes=False),
    scratch_types=[
        pltpu.VMEM((N,), jnp.int32),
        pltpu.VMEM((CHUNK,), jnp.int32),
        pltpu.VMEM((CHUNK, D), jnp.float32),
        pltpu.VMEM((ECHUNK,), jnp.int32),
        pltpu.SemaphoreType.DMA,
    ],
)


def kernel(x, edge_index, W, b):
    scores = pl.pallas_call(
        _scores_body,
        grid=(5,),
        in_specs=[
            pl.BlockSpec((2000, D), lambda i: (i, 0)),
            pl.BlockSpec((D, 1), lambda i: (0, 0)),
            pl.BlockSpec((1, 1), lambda i: (0, 0)),
        ],
        out_specs=pl.BlockSpec((2000, 1), lambda i: (i, 0)),
        out_shape=jax.ShapeDtypeStruct((N, 1), jnp.float32),
    )(x, W, b.reshape(1, 1))

    srow = jnp.pad(
        scores[:, 0], (0, JPAD - N), constant_values=-jnp.inf
    ).reshape(1, JPAD)

    rank_f = pl.pallas_call(
        _rank_body,
        grid=(N // _IB, JPAD // _JB),
        in_specs=[
            pl.BlockSpec((_IB, 1), lambda i, j: (i, 0)),
            pl.BlockSpec((1, _JB), lambda i, j: (0, j)),
        ],
        out_specs=pl.BlockSpec((_IB, 1), lambda i, j: (i, 0)),
        out_shape=jax.ShapeDtypeStruct((N, 1), jnp.float32),
    )(scores, srow)

    rank = rank_f[:, 0].astype(jnp.int32)
    perm, pooled, eout = _sc_call(rank, edge_index.reshape(-1), x)
    return pooled, eout.reshape(2, E2 // 2), perm


# speedup vs baseline: 30.4110x; 1.5328x over previous
import jax
import jax.numpy as jnp
from jax import lax
from jax.experimental import pallas as pl
from jax.experimental.pallas import tpu as pltpu
from jax.experimental.pallas import tpu_sc as plsc

N = 10000
NP = 10240
D = 128
K = 5000
NE = 320000
NW = 32
CHUNK = 160
ECHUNK = NE // 16
B = 1024


def _scores_body(x_ref, w_ref, b_ref, scol_ref, srow_ref):
    g = pl.program_id(0)
    s = (
        jnp.dot(x_ref[...], w_ref[...], preferred_element_type=jnp.float32)
        + b_ref[0, 0]
    )
    row = g * B + lax.broadcasted_iota(jnp.int32, (B, 1), 0)
    s = jnp.where(row < N, s, -jnp.inf)
    scol_ref[...] = s
    srow_ref[...] = s.T


def _rank_body(scol_ref, srow_ref, out_ref, acc_ref):
    bi = pl.program_id(0)
    bj = pl.program_id(1)
    si = scol_ref[...]
    sj = srow_ref[...]

    @pl.when(bj == 0)
    def _init():
        acc_ref[...] = jnp.zeros_like(acc_ref)

    @pl.when(bj < bi)
    def _below():
        acc_ref[...] += jnp.sum(
            jnp.where(sj >= si, 1.0, 0.0), axis=1, keepdims=True
        )

    @pl.when(bj > bi)
    def _above():
        acc_ref[...] += jnp.sum(
            jnp.where(sj > si, 1.0, 0.0), axis=1, keepdims=True
        )

    @pl.when(bj == bi)
    def _diag():
        ii = lax.broadcasted_iota(jnp.int32, (B, 1), 0)
        jj = lax.broadcasted_iota(jnp.int32, (1, B), 1)
        before = (sj > si) | ((sj == si) & (jj < ii))
        acc_ref[...] += jnp.sum(
            jnp.where(before, 1.0, 0.0), axis=1, keepdims=True
        )

    @pl.when(bj == NP // B - 1)
    def _emit():
        out_ref[...] = acc_ref[...].T.astype(jnp.int32)


def _sc_body(rank_hbm, edge_hbm, x_hbm, perm_hbm, pooled_hbm, eout_hbm,
             rank_v, perm_v, rows_v, ebuf, sem_e, sem_g):
    wid = lax.axis_index("s") * 2 + lax.axis_index("c")
    lo = wid * CHUNK

    pltpu.sync_copy(rank_hbm.at[0], rank_v)

    erow = wid // 16
    ebase = (wid % 16) * ECHUNK

    @pl.when(erow == 0)
    def _estart0():
        pltpu.make_async_copy(
            edge_hbm.at[pl.ds(ebase, ECHUNK)], ebuf, sem_e
        ).start()

    @pl.when(erow == 1)
    def _estart1():
        pltpu.make_async_copy(
            edge_hbm.at[pl.ds(NE + ebase, ECHUNK)], ebuf, sem_e
        ).start()

    def body_a(t, carry):
        r = rank_v[pl.ds(t * 16, 16)]
        m = (r >= lo) & (r < lo + CHUNK)
        vals = t * 16 + lax.iota(jnp.int32, 16)
        plsc.store_scatter(perm_v, [r - lo], vals, mask=m)
        return carry

    lax.fori_loop(0, NP // 16, body_a, 0)

    g1 = pltpu.make_async_copy(
        x_hbm.at[perm_v.at[pl.ds(0, 80)]], rows_v.at[pl.ds(0, 80)], sem_g
    )
    g2 = pltpu.make_async_copy(
        x_hbm.at[perm_v.at[pl.ds(80, 80)]], rows_v.at[pl.ds(80, 80)], sem_g
    )
    g1.start()
    g2.start()

    @pl.when(erow == 0)
    def _ewait0():
        pltpu.make_async_copy(
            edge_hbm.at[pl.ds(ebase, ECHUNK)], ebuf, sem_e
        ).wait()

    @pl.when(erow == 1)
    def _ewait1():
        pltpu.make_async_copy(
            edge_hbm.at[pl.ds(NE + ebase, ECHUNK)], ebuf, sem_e
        ).wait()

    def body_c(t, carry):
        e = ebuf[pl.ds(t * 16, 16)]
        r = plsc.load_gather(rank_v, [e])
        ebuf[pl.ds(t * 16, 16)] = jnp.where(r < K, r, 0)
        return carry

    lax.fori_loop(0, ECHUNK // 16, body_c, 0)

    @pl.when(erow == 0)
    def _eout0():
        pltpu.sync_copy(ebuf, eout_hbm.at[pl.ds(ebase, ECHUNK)])

    @pl.when(erow == 1)
    def _eout1():
        pltpu.sync_copy(ebuf, eout_hbm.at[pl.ds(NE + ebase, ECHUNK)])

    g1.wait()
    g2.wait()

    @pl.when(wid < NW - 1)
    def _full():
        pltpu.sync_copy(perm_v, perm_hbm.at[pl.ds(lo, CHUNK)])
        pltpu.sync_copy(rows_v, pooled_hbm.at[pl.ds(lo, CHUNK)])

    @pl.when(wid == NW - 1)
    def _tail():
        pltpu.sync_copy(perm_v.at[pl.ds(0, 40)], perm_hbm.at[pl.ds(K - 40, 40)])
        pltpu.sync_copy(rows_v.at[pl.ds(0, 40)], pooled_hbm.at[pl.ds(K - 40, 40)])


_sc_call = pl.kernel(
    _sc_body,
    out_type=(
        jax.ShapeDtypeStruct((K,), jnp.int32),
        jax.ShapeDtypeStruct((K, D), jnp.float32),
        jax.ShapeDtypeStruct((2 * NE,), jnp.int32),
    ),
    mesh=plsc.VectorSubcoreMesh(core_axis_name="c", subcore_axis_name="s"),
    compiler_params=pltpu.CompilerParams(needs_layout_passes=False),
    scratch_types=[
        pltpu.VMEM((NP,), jnp.int32),
        pltpu.VMEM((CHUNK,), jnp.int32),
        pltpu.VMEM((CHUNK, D), jnp.float32),
        pltpu.VMEM((ECHUNK,), jnp.int32),
        pltpu.SemaphoreType.DMA,
        pltpu.SemaphoreType.DMA,
    ],
)


def kernel(x, edge_index, W, b):
    scol, srow = pl.pallas_call(
        _scores_body,
        grid=(NP // B,),
        in_specs=[
            pl.BlockSpec((B, D), lambda i: (i, 0)),
            pl.BlockSpec((D, 1), lambda i: (0, 0)),
            pl.BlockSpec((1, 1), lambda i: (0, 0)),
        ],
        out_specs=[
            pl.BlockSpec((B, 1), lambda i: (i, 0)),
            pl.BlockSpec((1, B), lambda i: (0, i)),
        ],
        out_shape=[
            jax.ShapeDtypeStruct((NP, 1), jnp.float32),
            jax.ShapeDtypeStruct((1, NP), jnp.float32),
        ],
    )(x, W, b.reshape(1, 1))

    rank = pl.pallas_call(
        _rank_body,
        grid=(NP // B, NP // B),
        in_specs=[
            pl.BlockSpec((B, 1), lambda i, j: (i, 0)),
            pl.BlockSpec((1, B), lambda i, j: (0, j)),
        ],
        out_specs=pl.BlockSpec((1, B), lambda i, j: (0, i)),
        out_shape=jax.ShapeDtypeStruct((1, NP), jnp.int32),
        scratch_shapes=[pltpu.VMEM((B, 1), jnp.float32)],
    )(scol, srow)

    perm, pooled, eout = _sc_call(rank, edge_index.reshape(-1), x)
    return pooled, eout.reshape(2, NE), perm


# speedup vs baseline: 33.1238x; 1.0892x over previous
import jax
import jax.numpy as jnp
from jax import lax
from jax.experimental import pallas as pl
from jax.experimental.pallas import tpu as pltpu
from jax.experimental.pallas import tpu_sc as plsc

N = 10000
NP = 10240
D = 128
K = 5000
NE = 320000
NW = 32
CHUNK = 160
ECHUNK = NE // 16
B = 1024


def _rank_body(x_ref, w_ref, b_ref, out_ref, scol_ref, srow_ref, acc_ref):
    bi = pl.program_id(0)
    bj = pl.program_id(1)

    @pl.when(bi == 0)
    def _scores():
        s = (
            jnp.dot(x_ref[...], w_ref[...], preferred_element_type=jnp.float32)
            + b_ref[0, 0]
        )
        row = bj * B + lax.broadcasted_iota(jnp.int32, (B, 1), 0)
        s = jnp.where(row < N, s, -jnp.inf)
        scol_ref[pl.ds(bj * B, B), :] = s
        srow_ref[:, pl.ds(bj * B, B)] = s.T

    si = scol_ref[pl.ds(bi * B, B), :]
    sj = srow_ref[:, pl.ds(bj * B, B)]

    @pl.when(bj == 0)
    def _init():
        acc_ref[...] = jnp.zeros_like(acc_ref)

    @pl.when(bj < bi)
    def _below():
        acc_ref[...] += jnp.sum(
            jnp.where(sj >= si, 1.0, 0.0), axis=1, keepdims=True
        )

    @pl.when(bj > bi)
    def _above():
        acc_ref[...] += jnp.sum(
            jnp.where(sj > si, 1.0, 0.0), axis=1, keepdims=True
        )

    @pl.when(bj == bi)
    def _diag():
        ii = lax.broadcasted_iota(jnp.int32, (B, 1), 0)
        jj = lax.broadcasted_iota(jnp.int32, (1, B), 1)
        before = (sj > si) | ((sj == si) & (jj < ii))
        acc_ref[...] += jnp.sum(
            jnp.where(before, 1.0, 0.0), axis=1, keepdims=True
        )

    @pl.when(bj == NP // B - 1)
    def _emit():
        out_ref[...] = acc_ref[...].T.astype(jnp.int32)


def _sc_body(rank_hbm, edge_hbm, x_hbm, perm_hbm, pooled_hbm, eout_hbm,
             rank_v, perm_v, rows_v, ebuf, sem_e, sem_g):
    wid = lax.axis_index("s") * 2 + lax.axis_index("c")
    lo = wid * CHUNK

    pltpu.sync_copy(rank_hbm.at[0], rank_v)

    erow = wid // 16
    ebase = (wid % 16) * ECHUNK

    @pl.when(erow == 0)
    def _estart0():
        pltpu.make_async_copy(
            edge_hbm.at[pl.ds(ebase, ECHUNK)], ebuf, sem_e
        ).start()

    @pl.when(erow == 1)
    def _estart1():
        pltpu.make_async_copy(
            edge_hbm.at[pl.ds(NE + ebase, ECHUNK)], ebuf, sem_e
        ).start()

    def body_a(t, carry):
        r = rank_v[pl.ds(t * 16, 16)]
        m = (r >= lo) & (r < lo + CHUNK)
        vals = t * 16 + lax.iota(jnp.int32, 16)
        plsc.store_scatter(perm_v, [r - lo], vals, mask=m)
        return carry

    lax.fori_loop(0, NP // 16, body_a, 0)

    g1 = pltpu.make_async_copy(
        x_hbm.at[perm_v.at[pl.ds(0, 80)]], rows_v.at[pl.ds(0, 80)], sem_g
    )
    g2 = pltpu.make_async_copy(
        x_hbm.at[perm_v.at[pl.ds(80, 80)]], rows_v.at[pl.ds(80, 80)], sem_g
    )
    g1.start()
    g2.start()

    @pl.when(erow == 0)
    def _ewait0():
        pltpu.make_async_copy(
            edge_hbm.at[pl.ds(ebase, ECHUNK)], ebuf, sem_e
        ).wait()

    @pl.when(erow == 1)
    def _ewait1():
        pltpu.make_async_copy(
            edge_hbm.at[pl.ds(NE + ebase, ECHUNK)], ebuf, sem_e
        ).wait()

    def body_c(t, carry):
        e = ebuf[pl.ds(t * 16, 16)]
        r = plsc.load_gather(rank_v, [e])
        ebuf[pl.ds(t * 16, 16)] = jnp.where(r < K, r, 0)
        return carry

    lax.fori_loop(0, ECHUNK // 16, body_c, 0)

    @pl.when(erow == 0)
    def _eout0():
        pltpu.sync_copy(ebuf, eout_hbm.at[pl.ds(ebase, ECHUNK)])

    @pl.when(erow == 1)
    def _eout1():
        pltpu.sync_copy(ebuf, eout_hbm.at[pl.ds(NE + ebase, ECHUNK)])

    g1.wait()
    g2.wait()

    @pl.when(wid < NW - 1)
    def _full():
        pltpu.sync_copy(perm_v, perm_hbm.at[pl.ds(lo, CHUNK)])
        pltpu.sync_copy(rows_v, pooled_hbm.at[pl.ds(lo, CHUNK)])

    @pl.when(wid == NW - 1)
    def _tail():
        pltpu.sync_copy(perm_v.at[pl.ds(0, 40)], perm_hbm.at[pl.ds(K - 40, 40)])
        pltpu.sync_copy(rows_v.at[pl.ds(0, 40)], pooled_hbm.at[pl.ds(K - 40, 40)])


_sc_call = pl.kernel(
    _sc_body,
    out_type=(
        jax.ShapeDtypeStruct((K,), jnp.int32),
        jax.ShapeDtypeStruct((K, D), jnp.float32),
        jax.ShapeDtypeStruct((2 * NE,), jnp.int32),
    ),
    mesh=plsc.VectorSubcoreMesh(core_axis_name="c", subcore_axis_name="s"),
    compiler_params=pltpu.CompilerParams(needs_layout_passes=False),
    scratch_types=[
        pltpu.VMEM((NP,), jnp.int32),
        pltpu.VMEM((CHUNK,), jnp.int32),
        pltpu.VMEM((CHUNK, D), jnp.float32),
        pltpu.VMEM((ECHUNK,), jnp.int32),
        pltpu.SemaphoreType.DMA,
        pltpu.SemaphoreType.DMA,
    ],
)


def kernel(x, edge_index, W, b):
    rank = pl.pallas_call(
        _rank_body,
        grid=(NP // B, NP // B),
        in_specs=[
            pl.BlockSpec((B, D), lambda i, j: (jnp.where(i == 0, j, 0), 0)),
            pl.BlockSpec((D, 1), lambda i, j: (0, 0)),
            pl.BlockSpec((1, 1), lambda i, j: (0, 0)),
        ],
        out_specs=pl.BlockSpec((1, B), lambda i, j: (0, i)),
        out_shape=jax.ShapeDtypeStruct((1, NP), jnp.int32),
        scratch_shapes=[
            pltpu.VMEM((NP, 1), jnp.float32),
            pltpu.VMEM((1, NP), jnp.float32),
            pltpu.VMEM((B, 1), jnp.float32),
        ],
    )(x, W, b.reshape(1, 1))

    perm, pooled, eout = _sc_call(rank, edge_index.reshape(-1), x)
    return pooled, eout.reshape(2, NE), perm


# speedup vs baseline: 34.1539x; 1.0311x over previous
import jax
import jax.numpy as jnp
from jax import lax
from jax.experimental import pallas as pl
from jax.experimental.pallas import tpu as pltpu
from jax.experimental.pallas import tpu_sc as plsc

N = 10000
NP = 10240
D = 128
K = 5000
NE = 320000
NW = 32
CHUNK = 160
ECHUNK = NE // 16
B = 1024


def _rank_body(x_ref, w_ref, b_ref, out_ref, scol_ref, srow_ref, acc_ref,
               rowpart_ref, colacc_ref):
    bi = pl.program_id(0)
    bj = pl.program_id(1)

    @pl.when(bi == 0)
    def _scores():
        s = (
            jnp.dot(x_ref[...], w_ref[...], preferred_element_type=jnp.float32)
            + b_ref[0, 0]
        )
        row = bj * B + lax.broadcasted_iota(jnp.int32, (B, 1), 0)
        s = jnp.where(row < N, s, -jnp.inf)
        scol_ref[pl.ds(bj * B, B), :] = s
        srow_ref[:, pl.ds(bj * B, B)] = s.T

    si = scol_ref[pl.ds(bi * B, B), :]
    sj = srow_ref[:, pl.ds(bj * B, B)]
    ones_col = jnp.ones((B, 1), jnp.bfloat16)

    @pl.when(bj == 0)
    def _init():
        acc_ref[...] = jnp.zeros_like(acc_ref)

    @pl.when((bi == 0) & (bj == 0))
    def _initcol():
        colacc_ref[...] = jnp.zeros_like(colacc_ref)

    @pl.when(bj < bi)
    def _below():
        c = jnp.where(sj >= si, 1.0, 0.0)
        acc_ref[...] += jnp.dot(
            c.astype(jnp.bfloat16), ones_col,
            preferred_element_type=jnp.float32,
        )
        colacc_ref[:, pl.ds(bj * B, B)] += float(B) - jnp.sum(
            c, axis=0, keepdims=True
        )

    @pl.when(bj == bi)
    def _diag():
        ii = lax.broadcasted_iota(jnp.int32, (B, 1), 0)
        jj = lax.broadcasted_iota(jnp.int32, (1, B), 1)
        before = (sj > si) | ((sj == si) & (jj < ii))
        acc_ref[...] += jnp.dot(
            jnp.where(before, 1.0, 0.0).astype(jnp.bfloat16), ones_col,
            preferred_element_type=jnp.float32,
        )
        rowpart_ref[:, pl.ds(bi * B, B)] = acc_ref[...].T

    @pl.when((bi == NP // B - 1) & (bj == NP // B - 1))
    def _emit():
        out_ref[...] = (rowpart_ref[...] + colacc_ref[...]).astype(jnp.int32)


def _sc_body(rank_hbm, edge_hbm, x_hbm, perm_hbm, pooled_hbm, eout_hbm,
             rank_v, perm_v, rows_v, ebuf, sem_e, sem_g):
    wid = lax.axis_index("s") * 2 + lax.axis_index("c")
    lo = wid * CHUNK

    pltpu.sync_copy(rank_hbm.at[0], rank_v)

    erow = wid // 16
    ebase = (wid % 16) * ECHUNK

    @pl.when(erow == 0)
    def _estart0():
        pltpu.make_async_copy(
            edge_hbm.at[pl.ds(ebase, ECHUNK)], ebuf, sem_e
        ).start()

    @pl.when(erow == 1)
    def _estart1():
        pltpu.make_async_copy(
            edge_hbm.at[pl.ds(NE + ebase, ECHUNK)], ebuf, sem_e
        ).start()

    def body_a(t, carry):
        r = rank_v[pl.ds(t * 16, 16)]
        m = (r >= lo) & (r < lo + CHUNK)
        vals = t * 16 + lax.iota(jnp.int32, 16)
        plsc.store_scatter(perm_v, [r - lo], vals, mask=m)
        return carry

    lax.fori_loop(0, NP // 16, body_a, 0)

    g1 = pltpu.make_async_copy(
        x_hbm.at[perm_v.at[pl.ds(0, 80)]], rows_v.at[pl.ds(0, 80)], sem_g
    )
    g2 = pltpu.make_async_copy(
        x_hbm.at[perm_v.at[pl.ds(80, 80)]], rows_v.at[pl.ds(80, 80)], sem_g
    )
    g1.start()
    g2.start()

    @pl.when(erow == 0)
    def _ewait0():
        pltpu.make_async_copy(
            edge_hbm.at[pl.ds(ebase, ECHUNK)], ebuf, sem_e
        ).wait()

    @pl.when(erow == 1)
    def _ewait1():
        pltpu.make_async_copy(
            edge_hbm.at[pl.ds(NE + ebase, ECHUNK)], ebuf, sem_e
        ).wait()

    def body_c(t, carry):
        e = ebuf[pl.ds(t * 16, 16)]
        r = plsc.load_gather(rank_v, [e])
        ebuf[pl.ds(t * 16, 16)] = jnp.where(r < K, r, 0)
        return carry

    lax.fori_loop(0, ECHUNK // 16, body_c, 0)

    @pl.when(erow == 0)
    def _eout0():
        pltpu.sync_copy(ebuf, eout_hbm.at[pl.ds(ebase, ECHUNK)])

    @pl.when(erow == 1)
    def _eout1():
        pltpu.sync_copy(ebuf, eout_hbm.at[pl.ds(NE + ebase, ECHUNK)])

    g1.wait()
    g2.wait()

    @pl.when(wid < NW - 1)
    def _full():
        pltpu.sync_copy(perm_v, perm_hbm.at[pl.ds(lo, CHUNK)])
        pltpu.sync_copy(rows_v, pooled_hbm.at[pl.ds(lo, CHUNK)])

    @pl.when(wid == NW - 1)
    def _tail():
        pltpu.sync_copy(perm_v.at[pl.ds(0, 40)], perm_hbm.at[pl.ds(K - 40, 40)])
        pltpu.sync_copy(rows_v.at[pl.ds(0, 40)], pooled_hbm.at[pl.ds(K - 40, 40)])


_sc_call = pl.kernel(
    _sc_body,
    out_type=(
        jax.ShapeDtypeStruct((K,), jnp.int32),
        jax.ShapeDtypeStruct((K, D), jnp.float32),
        jax.ShapeDtypeStruct((2 * NE,), jnp.int32),
    ),
    mesh=plsc.VectorSubcoreMesh(core_axis_name="c", subcore_axis_name="s"),
    compiler_params=pltpu.CompilerParams(needs_layout_passes=False),
    scratch_types=[
        pltpu.VMEM((NP,), jnp.int32),
        pltpu.VMEM((CHUNK,), jnp.int32),
        pltpu.VMEM((CHUNK, D), jnp.float32),
        pltpu.VMEM((ECHUNK,), jnp.int32),
        pltpu.SemaphoreType.DMA,
        pltpu.SemaphoreType.DMA,
    ],
)


def kernel(x, edge_index, W, b):
    rank = pl.pallas_call(
        _rank_body,
        grid=(NP // B, NP // B),
        in_specs=[
            pl.BlockSpec((B, D), lambda i, j: (jnp.where(i == 0, j, 0), 0)),
            pl.BlockSpec((D, 1), lambda i, j: (0, 0)),
            pl.BlockSpec((1, 1), lambda i, j: (0, 0)),
        ],
        out_specs=pl.BlockSpec((1, NP), lambda i, j: (0, 0)),
        out_shape=jax.ShapeDtypeStruct((1, NP), jnp.int32),
        scratch_shapes=[
            pltpu.VMEM((NP, 1), jnp.float32),
            pltpu.VMEM((1, NP), jnp.float32),
            pltpu.VMEM((B, 1), jnp.float32),
            pltpu.VMEM((1, NP), jnp.float32),
            pltpu.VMEM((1, NP), jnp.float32),
        ],
    )(x, W, b.reshape(1, 1))

    perm, pooled, eout = _sc_call(rank, edge_index.reshape(-1), x)
    return pooled, eout.reshape(2, NE), perm


# speedup vs baseline: 39.1847x; 1.1473x over previous
import jax
import jax.numpy as jnp
from jax import lax
from jax.experimental import pallas as pl
from jax.experimental.pallas import tpu as pltpu
from jax.experimental.pallas import tpu_sc as plsc

N = 10000
NP = 10240
D = 128
K = 5000
NE = 320000
NW = 32
CHUNK = 160
ECHUNK = 9984
ETAIL = NE - (NW - 1) * ECHUNK
B = 1024


def _rank_body(x_ref, w_ref, b_ref, out_ref, scol_ref, srow_ref, acc_ref,
               rowpart_ref, colacc_ref):
    bi = pl.program_id(0)
    bj = pl.program_id(1)

    @pl.when(bi == 0)
    def _scores():
        s = (
            jnp.dot(x_ref[...], w_ref[...], preferred_element_type=jnp.float32)
            + b_ref[0, 0]
        )
        row = bj * B + lax.broadcasted_iota(jnp.int32, (B, 1), 0)
        s = jnp.where(row < N, s, -jnp.inf)
        scol_ref[pl.ds(bj * B, B), :] = s
        srow_ref[:, pl.ds(bj * B, B)] = s.T

    si = scol_ref[pl.ds(bi * B, B), :]
    sj = srow_ref[:, pl.ds(bj * B, B)]
    ones_col = jnp.ones((B, 1), jnp.bfloat16)

    @pl.when(bj == 0)
    def _init():
        acc_ref[...] = jnp.zeros_like(acc_ref)

    @pl.when((bi == 0) & (bj == 0))
    def _initcol():
        colacc_ref[...] = jnp.zeros_like(colacc_ref)

    @pl.when(bj < bi)
    def _below():
        c = jnp.where(sj >= si, 1.0, 0.0)
        acc_ref[...] += jnp.dot(
            c.astype(jnp.bfloat16), ones_col,
            preferred_element_type=jnp.float32,
        )
        colacc_ref[:, pl.ds(bj * B, B)] += float(B) - jnp.sum(
            c, axis=0, keepdims=True
        )

    @pl.when(bj == bi)
    def _diag():
        ii = lax.broadcasted_iota(jnp.int32, (B, 1), 0)
        jj = lax.broadcasted_iota(jnp.int32, (1, B), 1)
        before = (sj > si) | ((sj == si) & (jj < ii))
        acc_ref[...] += jnp.dot(
            jnp.where(before, 1.0, 0.0).astype(jnp.bfloat16), ones_col,
            preferred_element_type=jnp.float32,
        )
        rowpart_ref[:, pl.ds(bi * B, B)] = acc_ref[...].T

    @pl.when((bi == NP // B - 1) & (bj == NP // B - 1))
    def _emit():
        out_ref[...] = (rowpart_ref[...] + colacc_ref[...]).astype(jnp.int32)


def _sc_body(rank_hbm, edge_hbm, x_hbm, perm_hbm, pooled_hbm, eout_hbm,
             rank_v, perm_v, rows_v, ebuf, sem_e, sem_g):
    wid = lax.axis_index("s") * 2 + lax.axis_index("c")
    lo = wid * CHUNK

    pltpu.sync_copy(rank_hbm.at[0], rank_v)

    ebase = wid * ECHUNK

    @pl.when(wid < NW - 1)
    def _estart():
        pltpu.make_async_copy(
            edge_hbm.at[:, pl.ds(ebase, ECHUNK)],
            ebuf.at[:, pl.ds(0, ECHUNK)], sem_e,
        ).start()

    @pl.when(wid == NW - 1)
    def _estart_tail():
        pltpu.make_async_copy(
            edge_hbm.at[:, pl.ds(NE - ETAIL, ETAIL)], ebuf, sem_e
        ).start()

    def body_a(t, carry):
        r = rank_v[pl.ds(t * 16, 16)]
        m = (r >= lo) & (r < lo + CHUNK)
        vals = t * 16 + lax.iota(jnp.int32, 16)
        plsc.store_scatter(perm_v, [r - lo], vals, mask=m)
        return carry

    lax.fori_loop(0, NP // 16, body_a, 0)

    g1 = pltpu.make_async_copy(
        x_hbm.at[perm_v.at[pl.ds(0, 80)]], rows_v.at[pl.ds(0, 80)], sem_g
    )
    g2 = pltpu.make_async_copy(
        x_hbm.at[perm_v.at[pl.ds(80, 80)]], rows_v.at[pl.ds(80, 80)], sem_g
    )
    g1.start()
    g2.start()

    @pl.when(wid < NW - 1)
    def _ewait():
        pltpu.make_async_copy(
            edge_hbm.at[:, pl.ds(ebase, ECHUNK)],
            ebuf.at[:, pl.ds(0, ECHUNK)], sem_e,
        ).wait()

    @pl.when(wid == NW - 1)
    def _ewait_tail():
        pltpu.make_async_copy(
            edge_hbm.at[:, pl.ds(NE - ETAIL, ETAIL)], ebuf, sem_e
        ).wait()

    def body_c(t, carry):
        e0 = ebuf[0, pl.ds(t * 16, 16)]
        r0 = plsc.load_gather(rank_v, [e0])
        ebuf[0, pl.ds(t * 16, 16)] = jnp.where(r0 < K, r0, 0)
        e1 = ebuf[1, pl.ds(t * 16, 16)]
        r1 = plsc.load_gather(rank_v, [e1])
        ebuf[1, pl.ds(t * 16, 16)] = jnp.where(r1 < K, r1, 0)
        return carry

    nsteps = jnp.where(wid == NW - 1, ETAIL // 16, ECHUNK // 16)
    lax.fori_loop(0, nsteps, body_c, 0)

    @pl.when(wid < NW - 1)
    def _eout():
        pltpu.sync_copy(
            ebuf.at[:, pl.ds(0, ECHUNK)], eout_hbm.at[:, pl.ds(ebase, ECHUNK)]
        )

    @pl.when(wid == NW - 1)
    def _eout_tail():
        pltpu.sync_copy(ebuf, eout_hbm.at[:, pl.ds(NE - ETAIL, ETAIL)])

    g1.wait()
    g2.wait()

    @pl.when(wid < NW - 1)
    def _full():
        pltpu.sync_copy(perm_v, perm_hbm.at[pl.ds(lo, CHUNK)])
        pltpu.sync_copy(rows_v, pooled_hbm.at[pl.ds(lo, CHUNK)])

    @pl.when(wid == NW - 1)
    def _tail():
        pltpu.sync_copy(perm_v.at[pl.ds(0, 40)], perm_hbm.at[pl.ds(K - 40, 40)])
        pltpu.sync_copy(rows_v.at[pl.ds(0, 40)], pooled_hbm.at[pl.ds(K - 40, 40)])


_sc_call = pl.kernel(
    _sc_body,
    out_type=(
        jax.ShapeDtypeStruct((K,), jnp.int32),
        jax.ShapeDtypeStruct((K, D), jnp.float32),
        jax.ShapeDtypeStruct((2, NE), jnp.int32),
    ),
    mesh=plsc.VectorSubcoreMesh(core_axis_name="c", subcore_axis_name="s"),
    compiler_params=pltpu.CompilerParams(needs_layout_passes=False),
    scratch_types=[
        pltpu.VMEM((NP,), jnp.int32),
        pltpu.VMEM((CHUNK,), jnp.int32),
        pltpu.VMEM((CHUNK, D), jnp.float32),
        pltpu.VMEM((2, ETAIL), jnp.int32),
        pltpu.SemaphoreType.DMA,
        pltpu.SemaphoreType.DMA,
    ],
)


def kernel(x, edge_index, W, b):
    rank = pl.pallas_call(
        _rank_body,
        grid=(NP // B, NP // B),
        in_specs=[
            pl.BlockSpec((B, D), lambda i, j: (jnp.where(i == 0, j, 0), 0)),
            pl.BlockSpec((D, 1), lambda i, j: (0, 0)),
            pl.BlockSpec((1, 1), lambda i, j: (0, 0)),
        ],
        out_specs=pl.BlockSpec((1, NP), lambda i, j: (0, 0)),
        out_shape=jax.ShapeDtypeStruct((1, NP), jnp.int32),
        scratch_shapes=[
            pltpu.VMEM((NP, 1), jnp.float32),
            pltpu.VMEM((1, NP), jnp.float32),
            pltpu.VMEM((B, 1), jnp.float32),
            pltpu.VMEM((1, NP), jnp.float32),
            pltpu.VMEM((1, NP), jnp.float32),
        ],
    )(x, W, b.reshape(1, 1))

    perm, pooled, eout = _sc_call(rank, edge_index, x)
    return pooled, eout, perm
